# baseline (speedup 1.0000x reference)
import functools

import jax
import jax.numpy as jnp
from jax import lax
from jax.experimental import pallas as pl
from jax.experimental.pallas import tpu as pltpu
from jax.experimental.pallas import tpu_sc as plsc

VOCAB = 1000000
DIM = 32
N_NODES = 100000

_BR = 2048
_BC = 4 * _BR
_NBLK = -(-VOCAB // _BC)
_PROWS = _NBLK * _BR

_info = plsc.get_sparse_core_info()
_NC, _NS = _info.num_cores, _info.num_subcores
_NW = _NC * _NS

_CHUNK = 640
_NCHUNK = 5
_B_PER_W = _CHUNK * _NCHUNK
_B_PAD = _B_PER_W * _NW


_NFULL = VOCAB // _BC


def _tc_repack(b0_ref, b1_ref, b2_ref, b3_ref, t0_ref, t1_ref, t2_ref,
               t3_ref, out_ref):
    i = pl.program_id(0)

    @pl.when(i < _NFULL)
    def _main():
        out_ref[...] = jnp.concatenate(
            [r[...].T for r in (b0_ref, b1_ref, b2_ref, b3_ref)], axis=1)

    @pl.when(i == _NFULL)
    def _tail():
        out_ref[...] = jnp.concatenate(
            [r[...].T for r in (t0_ref, t1_ref, t2_ref, t3_ref)], axis=1)


_repack = pl.pallas_call(
    _tc_repack,
    grid=(_NBLK,),
    in_specs=[
        pl.BlockSpec(
            (DIM, _BR),
            functools.partial(
                lambda s, i: (0, jnp.where(i < _NFULL, 4 * i + s, 0)), s))
        for s in range(4)
    ] + [
        pl.BlockSpec(
            (DIM, _BR),
            functools.partial(
                lambda s, i: (0, jnp.where(i < _NFULL, 0, s)), s))
        for s in range(4)
    ],
    out_specs=pl.BlockSpec((_BR, 128), lambda i: (i, 0)),
    out_shape=jax.ShapeDtypeStruct((_PROWS, 128), jnp.float32),
)


def _make_sc_gather():
    mesh = plsc.VectorSubcoreMesh(core_axis_name="c", subcore_axis_name="s")

    @functools.partial(
        pl.kernel,
        mesh=mesh,
        out_type=jax.ShapeDtypeStruct((_B_PAD, 128), jnp.float32),
        scratch_types=[
            pltpu.VMEM((_CHUNK,), jnp.int32),
            pltpu.VMEM((_CHUNK, 128), jnp.float32),
            pltpu.SemaphoreType.DMA,
        ],
        compiler_params=pltpu.CompilerParams(use_tc_tiling_on_sc=False),
    )
    def sc_gather(packed_hbm, rows_hbm, out_hbm, row_v, wide_v, sem):
        wid = lax.axis_index("s") * _NC + lax.axis_index("c")
        base = wid * _B_PER_W

        def do_chunk(chunk_base):
            pltpu.sync_copy(rows_hbm.at[pl.ds(chunk_base, _CHUNK)], row_v)
            pltpu.async_copy(packed_hbm.at[row_v], wide_v, sem).wait()
            pltpu.sync_copy(wide_v, out_hbm.at[pl.ds(chunk_base, _CHUNK)])

        for k in range(_NCHUNK):
            do_chunk(base + k * _CHUNK)

    return sc_gather


_sc_gather = _make_sc_gather()


def kernel(token_id, table):
    idx = jnp.pad(token_id, (0, _B_PAD - N_NODES))
    rows = ((idx >> 13) << 11) + (idx & 2047)
    tt = table.T
    tail_tt = jnp.pad(tt[:, _NFULL * _BC:], ((0, 0), (0, (_NFULL + 1) * _BC - VOCAB)))
    packed = _repack(tt, tt, tt, tt, tail_tt, tail_tt, tail_tt, tail_tt)
    wide = _sc_gather(packed, rows)
    band = ((idx[:N_NODES] >> 11) & 3)[:, None]
    w = wide[:N_NODES]
    out = jnp.where(band == 0, w[:, 0:32], 0.0)
    out = out + jnp.where(band == 1, w[:, 32:64], 0.0)
    out = out + jnp.where(band == 2, w[:, 64:96], 0.0)
    out = out + jnp.where(band == 3, w[:, 96:128], 0.0)
    return out


# speedup vs baseline: 1.2590x; 1.2590x over previous
import functools

import jax
import jax.numpy as jnp
from jax import lax
from jax.experimental import pallas as pl
from jax.experimental.pallas import tpu as pltpu
from jax.experimental.pallas import tpu_sc as plsc

VOCAB = 1000000
DIM = 32
N_NODES = 100000

_BR = 2048
_BC = 4 * _BR
_NBLK = -(-VOCAB // _BC)
_PROWS = _NBLK * _BR

_info = plsc.get_sparse_core_info()
_NC, _NS = _info.num_cores, _info.num_subcores
_NW = _NC * _NS

_CHUNK = 640
_NCHUNK = 5
_B_PER_W = _CHUNK * _NCHUNK
_B_PAD = _B_PER_W * _NW


_NFULL = VOCAB // _BC


def _tc_repack(b0_ref, b1_ref, b2_ref, b3_ref, t0_ref, t1_ref, t2_ref,
               t3_ref, out_ref):
    i = pl.program_id(0)

    @pl.when(i < _NFULL)
    def _main():
        out_ref[...] = jnp.concatenate(
            [b0_ref[...], b1_ref[...], b2_ref[...], b3_ref[...]], axis=0).T

    @pl.when(i == _NFULL)
    def _tail():
        out_ref[...] = jnp.concatenate(
            [t0_ref[...], t1_ref[...], t2_ref[...], t3_ref[...]], axis=0).T


_repack = pl.pallas_call(
    _tc_repack,
    grid=(_NBLK,),
    in_specs=[
        pl.BlockSpec(
            (DIM, _BR),
            functools.partial(
                lambda s, i: (0, jnp.where(i < _NFULL, 4 * i + s, 0)), s))
        for s in range(4)
    ] + [
        pl.BlockSpec(
            (DIM, _BR),
            functools.partial(
                lambda s, i: (0, jnp.where(i < _NFULL, 0, s)), s))
        for s in range(4)
    ],
    out_specs=pl.BlockSpec((_BR, 128), lambda i: (i, 0)),
    out_shape=jax.ShapeDtypeStruct((_PROWS, 128), jnp.float32),
)


def _make_sc_gather():
    mesh = plsc.VectorSubcoreMesh(core_axis_name="c", subcore_axis_name="s")

    @functools.partial(
        pl.kernel,
        mesh=mesh,
        out_type=jax.ShapeDtypeStruct((_B_PAD, 128), jnp.float32),
        scratch_types=[
            pltpu.VMEM((_CHUNK,), jnp.int32),
            pltpu.VMEM((_CHUNK, 128), jnp.float32),
            pltpu.SemaphoreType.DMA,
        ],
        compiler_params=pltpu.CompilerParams(use_tc_tiling_on_sc=False),
    )
    def sc_gather(packed_hbm, rows_hbm, out_hbm, row_v, wide_v, sem):
        wid = lax.axis_index("s") * _NC + lax.axis_index("c")
        base = wid * _B_PER_W

        def do_chunk(chunk_base):
            pltpu.sync_copy(rows_hbm.at[pl.ds(chunk_base, _CHUNK)], row_v)
            pltpu.async_copy(packed_hbm.at[row_v], wide_v, sem).wait()
            pltpu.sync_copy(wide_v, out_hbm.at[pl.ds(chunk_base, _CHUNK)])

        for k in range(_NCHUNK):
            do_chunk(base + k * _CHUNK)

    return sc_gather


_sc_gather = _make_sc_gather()


def kernel(token_id, table):
    idx = jnp.pad(token_id, (0, _B_PAD - N_NODES))
    rows = ((idx >> 13) << 11) + (idx & 2047)
    tt = table.T
    tail_tt = jnp.pad(tt[:, _NFULL * _BC:], ((0, 0), (0, (_NFULL + 1) * _BC - VOCAB)))
    packed = _repack(tt, tt, tt, tt, tail_tt, tail_tt, tail_tt, tail_tt)
    wide = _sc_gather(packed, rows)
    band = ((idx[:N_NODES] >> 11) & 3)[:, None]
    w = wide[:N_NODES]
    out = jnp.where(band == 0, w[:, 0:32], 0.0)
    out = out + jnp.where(band == 1, w[:, 32:64], 0.0)
    out = out + jnp.where(band == 2, w[:, 64:96], 0.0)
    out = out + jnp.where(band == 3, w[:, 96:128], 0.0)
    return out
